# baseline (speedup 1.0000x reference)
import jax
import jax.numpy as jnp
from jax import lax
from jax.experimental import pallas as pl
from jax.experimental.pallas import tpu as pltpu
from jax.experimental.pallas import tpu_sc as plsc

BATCH = 16384
HIST = 50
EMB_DIM = 64
TOTAL = BATCH * HIST

NUM_CORES = 2
NUM_SUBCORES = 16
NW = NUM_CORES * NUM_SUBCORES
B_PER_W = BATCH // NW
IDX_PER_W = B_PER_W * HIST
BB_PER_W = B_PER_W // 128
LANES = 16


def _body(x_hbm, table_hbm, out_hbm, xv, iv, rv, st, sem_g, sem_w):
    wid = lax.axis_index("s") * NUM_CORES + lax.axis_index("c")
    base = wid * IDX_PER_W
    bb0 = wid * BB_PER_W

    pltpu.sync_copy(x_hbm.at[pl.ds(base, IDX_PER_W)], xv)

    def per_h(h, carry):
        lane = lax.iota(jnp.int32, LANES)
        def build(k, c):
            pos = (k * LANES + lane) * HIST + h
            iv[pl.ds(k * LANES, LANES)] = plsc.load_gather(xv, [pos])
            return c

        lax.fori_loop(0, B_PER_W // LANES, build, 0)

        pltpu.async_copy(table_hbm.at[iv], rv, sem_g).wait()

        def transpose_col(t, c):
            cb = t // 8
            c8 = t - cb * 8
            col = lane * 0 + t
            for j in range(BB_PER_W):
                for k in range(128 // LANES):
                    row = j * 128 + k * LANES + lane
                    st[cb, j, c8, pl.ds(k * LANES, LANES)] = (
                        plsc.load_gather(rv, [row, col]))
            return c

        lax.fori_loop(0, EMB_DIM, transpose_col, 0)


        def write(cb, c):
            pltpu.async_copy(st.at[cb], out_hbm.at[h, cb, pl.ds(bb0, BB_PER_W)],
                             sem_w)
            return c

        lax.fori_loop(0, 8, write, 0)

        def drain(cb, c):
            pltpu.make_async_copy(
                st.at[cb], out_hbm.at[h, cb, pl.ds(bb0, BB_PER_W)], sem_w
            ).wait()
            return c

        lax.fori_loop(0, 8, drain, 0)
        return carry

    lax.fori_loop(0, HIST, per_h, 0)


@jax.jit
def _gather(x_flat, table):
    mesh = plsc.VectorSubcoreMesh(core_axis_name="c", subcore_axis_name="s")
    k = pl.kernel(
        _body,
        out_type=jax.ShapeDtypeStruct((HIST, 8, BATCH // 128, 8, 128),
                                      jnp.float32),
        mesh=mesh,
        compiler_params=pltpu.CompilerParams(
            use_tc_tiling_on_sc=False, needs_layout_passes=False),
        scratch_types=[
            pltpu.VMEM((IDX_PER_W,), jnp.int32),
            pltpu.VMEM((B_PER_W,), jnp.int32),
            pltpu.VMEM((B_PER_W, EMB_DIM), jnp.float32),
            pltpu.VMEM((8, BB_PER_W, 8, 128), jnp.float32),
            pltpu.SemaphoreType.DMA,
            pltpu.SemaphoreType.DMA,
        ],
    )
    return k(x_flat, table)


def kernel(x, table):
    x_flat = x.reshape(-1).astype(jnp.int32)
    out5 = _gather(x_flat, table)
    out = out5.transpose(2, 4, 0, 1, 3).reshape(BATCH, HIST, EMB_DIM)
    return out


# speedup vs baseline: 1.8472x; 1.8472x over previous
import jax
import jax.numpy as jnp
from jax import lax
from jax.experimental import pallas as pl
from jax.experimental.pallas import tpu as pltpu
from jax.experimental.pallas import tpu_sc as plsc

BATCH = 16384
HIST = 50
EMB_DIM = 64
TOTAL = BATCH * HIST

NUM_CORES = 2
NUM_SUBCORES = 16
NW = NUM_CORES * NUM_SUBCORES
B_PER_W = BATCH // NW
BB_PER_W = B_PER_W // 128
LANES = 16
N_PAIRS = HIST // 2


def _transpose_to_tiles(rv, st):
    lane = lax.iota(jnp.int32, LANES)

    def block(t, c):
        j = t // 32
        rem = t - j * 32
        c16 = rem // 8
        r16 = rem - c16 * 8
        r0 = r16 * LANES
        c0 = c16 * LANES
        rows = j * 128 + r0 + lane
        jv = lane * 0 + j
        blv = r0 + lane
        for d in range(LANES):
            colv = c0 + ((lane + d) & (LANES - 1))
            vals = plsc.load_gather(rv, [rows, colv])
            plsc.store_scatter(
                st, [colv >> 3, jv, colv & 7, blv], vals)
        return c

    lax.fori_loop(0, 128, block, 0)


def _body(xt_hbm, table_hbm, out_hbm, iv_all, rv_a, rv_b, st,
          sem_i, sem_a, sem_b, sem_w):
    wid = lax.axis_index("s") * NUM_CORES + lax.axis_index("c")
    b0 = wid * B_PER_W
    bb0 = wid * BB_PER_W

    pltpu.async_copy(xt_hbm.at[:, pl.ds(b0, B_PER_W)], iv_all, sem_i).wait()

    def writes(h, start):
        def w(cb, c):
            cpy = pltpu.make_async_copy(
                st.at[cb], out_hbm.at[h, cb, pl.ds(bb0, BB_PER_W)], sem_w)
            if start:
                cpy.start()
            else:
                cpy.wait()
            return c
        lax.fori_loop(0, 8, w, 0)

    def gather(h, rv, sem):
        pltpu.async_copy(table_hbm.at[iv_all.at[h]], rv, sem)

    def pair(p, carry):
        h0 = 2 * p
        h1 = h0 + 1

        @pl.when(p == 0)
        def _():
            gather(h0, rv_a, sem_a)

        gather(h1, rv_b, sem_b)

        pltpu.make_async_copy(table_hbm.at[iv_all.at[h0]], rv_a, sem_a).wait()

        @pl.when(p > 0)
        def _():
            writes(h0 - 1, start=False)

        _transpose_to_tiles(rv_a, st)
        writes(h0, start=True)

        @pl.when(p < N_PAIRS - 1)
        def _():
            gather(h0 + 2, rv_a, sem_a)

        pltpu.make_async_copy(table_hbm.at[iv_all.at[h1]], rv_b, sem_b).wait()
        writes(h0, start=False)
        _transpose_to_tiles(rv_b, st)
        writes(h1, start=True)
        return carry

    lax.fori_loop(0, N_PAIRS, pair, 0)
    writes(HIST - 1, start=False)


@jax.jit
def _gather(x_t, table):
    mesh = plsc.VectorSubcoreMesh(core_axis_name="c", subcore_axis_name="s")
    k = pl.kernel(
        _body,
        out_type=jax.ShapeDtypeStruct((HIST, 8, BATCH // 128, 8, 128),
                                      jnp.float32),
        mesh=mesh,
        compiler_params=pltpu.CompilerParams(
            use_tc_tiling_on_sc=False, needs_layout_passes=False),
        scratch_types=[
            pltpu.VMEM((HIST, B_PER_W), jnp.int32),
            pltpu.VMEM((B_PER_W, EMB_DIM), jnp.float32),
            pltpu.VMEM((B_PER_W, EMB_DIM), jnp.float32),
            pltpu.VMEM((8, BB_PER_W, 8, 128), jnp.float32),
            pltpu.SemaphoreType.DMA,
            pltpu.SemaphoreType.DMA,
            pltpu.SemaphoreType.DMA,
            pltpu.SemaphoreType.DMA,
        ],
    )
    return k(x_t, table)


def kernel(x, table):
    x_t = x.T.astype(jnp.int32)
    out5 = _gather(x_t, table)
    out = out5.transpose(2, 4, 0, 1, 3).reshape(BATCH, HIST, EMB_DIM)
    return out


# speedup vs baseline: 1.8500x; 1.0015x over previous
import jax
import jax.numpy as jnp
from jax import lax
from jax.experimental import pallas as pl
from jax.experimental.pallas import tpu as pltpu
from jax.experimental.pallas import tpu_sc as plsc

BATCH = 16384
HIST = 50
EMB_DIM = 64
TOTAL = BATCH * HIST

NUM_CORES = 2
NUM_SUBCORES = 16
NW = NUM_CORES * NUM_SUBCORES
B_PER_W = BATCH // NW
BB_PER_W = B_PER_W // 128
LANES = 16
N_PAIRS = HIST // 2


def _transpose_to_tiles(rv, st):
    lane = lax.iota(jnp.int32, LANES)

    def block(t, c):
        j = t // 32
        rem = t - j * 32
        c16 = rem // 8
        r16 = rem - c16 * 8
        r0 = r16 * LANES
        c0 = c16 * LANES
        rows = j * 128 + r0 + lane
        jv = lane * 0 + j
        blv = r0 + lane
        for d in range(LANES):
            colv = c0 + ((lane + d) & (LANES - 1))
            vals = plsc.load_gather(rv, [rows, colv])
            plsc.store_scatter(
                st, [colv >> 3, jv, colv & 7, blv], vals)
        return c

    lax.fori_loop(0, 128, block, 0)


HALF = B_PER_W // 2


def _body(xt_hbm, table_hbm, out_hbm, iv_all, rv_a, rv_b, st,
          sem_i, sem_a, sem_a2, sem_b, sem_b2, sem_w):
    wid = lax.axis_index("s") * NUM_CORES + lax.axis_index("c")
    b0 = wid * B_PER_W
    bb0 = wid * BB_PER_W

    pltpu.async_copy(xt_hbm.at[:, pl.ds(b0, B_PER_W)], iv_all, sem_i).wait()

    def writes(h, start):
        def w(cb, c):
            cpy = pltpu.make_async_copy(
                st.at[cb], out_hbm.at[h, cb, pl.ds(bb0, BB_PER_W)], sem_w)
            if start:
                cpy.start()
            else:
                cpy.wait()
            return c
        lax.fori_loop(0, 8, w, 0)

    def gather(h, rv, sem, sem2):
        pltpu.async_copy(table_hbm.at[iv_all.at[h, pl.ds(0, HALF)]],
                         rv.at[pl.ds(0, HALF)], sem)
        pltpu.async_copy(table_hbm.at[iv_all.at[h, pl.ds(HALF, HALF)]],
                         rv.at[pl.ds(HALF, HALF)], sem2)

    def gwait(h, rv, sem, sem2):
        pltpu.make_async_copy(table_hbm.at[iv_all.at[h, pl.ds(0, HALF)]],
                              rv.at[pl.ds(0, HALF)], sem).wait()
        pltpu.make_async_copy(table_hbm.at[iv_all.at[h, pl.ds(HALF, HALF)]],
                              rv.at[pl.ds(HALF, HALF)], sem2).wait()

    def pair(p, carry):
        h0 = 2 * p
        h1 = h0 + 1

        @pl.when(p == 0)
        def _():
            gather(h0, rv_a, sem_a, sem_a2)

        gather(h1, rv_b, sem_b, sem_b2)

        gwait(h0, rv_a, sem_a, sem_a2)

        @pl.when(p > 0)
        def _():
            writes(h0 - 1, start=False)

        _transpose_to_tiles(rv_a, st)
        writes(h0, start=True)

        @pl.when(p < N_PAIRS - 1)
        def _():
            gather(h0 + 2, rv_a, sem_a, sem_a2)

        gwait(h1, rv_b, sem_b, sem_b2)
        writes(h0, start=False)
        _transpose_to_tiles(rv_b, st)
        writes(h1, start=True)
        return carry

    lax.fori_loop(0, N_PAIRS, pair, 0)
    writes(HIST - 1, start=False)


@jax.jit
def _gather(x_t, table):
    mesh = plsc.VectorSubcoreMesh(core_axis_name="c", subcore_axis_name="s")
    k = pl.kernel(
        _body,
        out_type=jax.ShapeDtypeStruct((HIST, 8, BATCH // 128, 8, 128),
                                      jnp.float32),
        mesh=mesh,
        compiler_params=pltpu.CompilerParams(
            use_tc_tiling_on_sc=False, needs_layout_passes=False),
        scratch_types=[
            pltpu.VMEM((HIST, B_PER_W), jnp.int32),
            pltpu.VMEM((B_PER_W, EMB_DIM), jnp.float32),
            pltpu.VMEM((B_PER_W, EMB_DIM), jnp.float32),
            pltpu.VMEM((8, BB_PER_W, 8, 128), jnp.float32),
            pltpu.SemaphoreType.DMA,
            pltpu.SemaphoreType.DMA,
            pltpu.SemaphoreType.DMA,
            pltpu.SemaphoreType.DMA,
            pltpu.SemaphoreType.DMA,
            pltpu.SemaphoreType.DMA,
        ],
    )
    return k(x_t, table)


def kernel(x, table):
    x_t = x.T.astype(jnp.int32)
    out5 = _gather(x_t, table)
    out = out5.transpose(2, 4, 0, 1, 3).reshape(BATCH, HIST, EMB_DIM)
    return out
